# baseline (speedup 1.0000x reference)
import functools

import jax
import jax.numpy as jnp
from jax import lax
from jax.experimental import pallas as pl
from jax.experimental.pallas import tpu as pltpu
from jax.experimental.pallas import tpu_sc as plsc


def _make_sc_broadcast(batch, seq_len, d_model, dtype):
    info = plsc.get_sparse_core_info()
    num_workers = info.num_cores * info.num_subcores
    rows_per_worker = seq_len // num_workers
    chunk = min(32, rows_per_worker)
    num_chunks = rows_per_worker // chunk

    mesh = plsc.VectorSubcoreMesh(core_axis_name="c", subcore_axis_name="s")

    @functools.partial(
        pl.kernel,
        mesh=mesh,
        out_type=jax.ShapeDtypeStruct((batch, seq_len, d_model), dtype),
        scratch_types=[
            pltpu.VMEM((chunk, d_model), dtype),
            pltpu.VMEM((chunk, d_model), dtype),
            pltpu.SemaphoreType.DMA,
            pltpu.SemaphoreType.DMA,
            pltpu.SemaphoreType.DMA,
            pltpu.SemaphoreType.DMA,
        ],
    )
    def sc_broadcast(table_hbm, out_hbm, buf0, buf1, rs0, rs1, ws0, ws1):
        wid = lax.axis_index("s") * info.num_cores + lax.axis_index("c")
        base = wid * rows_per_worker
        bufs = (buf0, buf1)
        rsems = (rs0, rs1)
        wsems = (ws0, ws1)

        def start_read(i):
            return pltpu.async_copy(
                table_hbm.at[pl.ds(base + i * chunk, chunk)],
                bufs[i % 2],
                rsems[i % 2],
            )

        reads = [None] * num_chunks
        writes = [None] * num_chunks
        reads[0] = start_read(0)
        for i in range(num_chunks):
            reads[i].wait()
            if i >= 1:
                for h in writes[i - 1]:
                    h.wait()
            if i + 1 < num_chunks:
                reads[i + 1] = start_read(i + 1)
            writes[i] = [
                pltpu.async_copy(
                    bufs[i % 2],
                    out_hbm.at[b, pl.ds(base + i * chunk, chunk)],
                    wsems[i % 2],
                )
                for b in range(batch)
            ]
        for h in writes[num_chunks - 1]:
            h.wait()

    return sc_broadcast


def kernel(x, table):
    batch, seq_len, d_model = x.shape
    fn = _make_sc_broadcast(batch, seq_len, d_model, table.dtype)
    return fn(table)
